# baseline (speedup 1.0000x reference)
import functools

import jax
import jax.numpy as jnp
import numpy as np
from jax import lax
from jax.experimental import pallas as pl
from jax.experimental.pallas import tpu as pltpu
from jax.experimental.pallas import tpu_sc as plsc

_B, _P, _C = 32, 20000, 3
_MSB = np.int32(-2147483648)
_NC, _NS, _L = 2, 16, 16
_VECS = _P // _L


def _tc_row_kernel(c0_ref, c1_ref, c2_ref, lab_ref, lab4_ref, pd_ref, gt_ref,
                   key_ref, npos_ref, pnll_ref, sl1_ref):
    c0 = c0_ref[0, 0, :]
    c1 = c1_ref[0, 0, :]
    c2 = c2_ref[0, 0, :]
    lab = lab_ref[0, 0, :]

    m = jnp.maximum(c0, jnp.maximum(c1, c2))
    lse = m + jnp.log(jnp.exp(c0 - m) + jnp.exp(c1 - m) + jnp.exp(c2 - m))
    mining = lse - c0
    pos = lab > 0

    bits = lax.bitcast_convert_type(mining, jnp.int32)
    u = jnp.where(bits >= 0, bits | _MSB, ~bits)
    u = jnp.where(pos, np.int32(0), u)
    key_ref[...] = u.reshape(1, 1, _P)

    npos = jnp.sum(pos.astype(jnp.int32))
    sel = jnp.where(lab == 1, c1, c2)
    pnll = jnp.sum(jnp.where(pos, lse - sel, 0.0))

    d = pd_ref[0, 0, :] - gt_ref[0, 0, :]
    ad = jnp.abs(d)
    sl1 = jnp.where(ad < 1.0, 0.5 * d * d, ad - 0.5)
    pos4 = lab4_ref[0, 0, :] > 0
    sl1_row = jnp.sum(jnp.where(pos4, sl1, 0.0))

    npos_ref[...] = jnp.full((1, 1, 128), npos, jnp.int32)
    pnll_ref[...] = jnp.full((1, 1, 128), pnll, jnp.float32)
    sl1_ref[...] = jnp.full((1, 1, 128), sl1_row, jnp.float32)


def _sc_topk_body(keys_hbm, npos_hbm, out_hbm,
                  keys_v, hc_v, cc_v, np_v, out_v):
    wid = lax.axis_index("s") * _NC + lax.axis_index("c")
    pltpu.sync_copy(keys_hbm.at[wid], keys_v)
    pltpu.sync_copy(npos_hbm, np_v)

    li = lax.iota(jnp.int32, 16)
    lane = wid % 16
    sel = jnp.where(wid >= 16, np_v[pl.ds(16, 16)], np_v[pl.ds(0, 16)])
    npos = jnp.sum(jnp.where(li == lane, sel, 0))
    k0 = jnp.minimum(3 * npos, _P - npos)

    ones = jnp.ones((16,), jnp.int32)
    zi = jnp.zeros((16,), jnp.int32)
    zf = jnp.zeros((16,), jnp.float32)

    def decode(v):
        return plsc.bitcast(jnp.where(v < 0, v & np.int32(0x7FFFFFFF), ~v),
                            jnp.float32)

    r = k0
    pfx = jnp.int32(0)

    for lvl in range(4):
        shift = 24 - 8 * lvl
        mhi = np.int32(0) if lvl == 0 else np.int32(
            np.uint32(0xFFFFFFFF) << np.uint32(shift + 8))

        def zero4(j, c):
            for q in range(4):
                jj = j * 4 + q
                hc_v[pl.ds(jj * 16, 16)] = zi
            return c
        lax.fori_loop(0, 64, zero4, 0)
        for g in range(16):
            cc_v[pl.ds(g * 16, 16)] = zi

        def dbody5(j, c, pfx=pfx, mhi=mhi, shift=shift, lvl=lvl):
            for q in range(5):
                v = keys_v[pl.ds((j * 5 + q) * 16, 16)]
                digit = lax.shift_right_logical(v, shift) & np.int32(255)
                idx = li * 256 + digit
                if lvl == 0:
                    plsc.addupdate_scatter(hc_v, [idx], ones)
                else:
                    msk = (v & mhi) == pfx
                    plsc.addupdate_scatter(hc_v, [idx], ones, mask=msk)
            return c
        lax.fori_loop(0, _VECS // 5, dbody5, 0)

        def fold2(j, c):
            for q in range(2):
                jj = j * 2 + q
                bidx = (jj & 15) * 16 + li
                plsc.addupdate_scatter(cc_v, [bidx], hc_v[pl.ds(jj * 16, 16)])
            return c
        lax.fori_loop(0, 128, fold2, 0)

        cgs = [cc_v[pl.ds(g * 16, 16)] for g in range(16)]
        tots = [jnp.sum(cg) for cg in cgs]
        hsum = jnp.int32(0)
        highs = [None] * 16
        for g in range(15, -1, -1):
            highs[g] = hsum
            hsum = hsum + tots[g]
        ntrue = jnp.int32(0)
        for g in range(16):
            a_incl = jnp.flip(jnp.cumsum(jnp.flip(cgs[g], 0)), 0)
            a = a_incl - cgs[g] + highs[g]
            ntrue = ntrue + jnp.sum(jnp.where(a < r, 1, 0))
        beta = np.int32(256) - ntrue
        cnt_above = jnp.int32(0)
        for g in range(16):
            mgt = (g * 16 + li) > beta
            cnt_above = cnt_above + jnp.sum(jnp.where(mgt, cgs[g], 0))
        r = r - cnt_above
        pfx = pfx | lax.shift_left(beta, np.int32(shift))

    tx = pfx ^ _MSB
    def sbody5(j, acc):
        sa = acc
        for q in range(5):
            v = keys_v[pl.ds((j * 5 + q) * 16, 16)]
            gt_m = (v ^ _MSB) > tx
            sa = sa + jnp.where(gt_m, decode(v), 0.0)
        return sa
    svec = lax.fori_loop(0, _VECS // 5, sbody5, zf)
    sum_gt = jnp.sum(svec)

    tv = jnp.full((16,), pfx, jnp.int32)
    ans = sum_gt + r.astype(jnp.float32) * decode(tv)
    ans = jnp.where(k0 > 0, ans, 0.0)
    out_v[...] = ans
    pltpu.sync_copy(out_v, out_hbm.at[wid])


_sc_topk = pl.kernel(
    _sc_topk_body,
    out_type=jax.ShapeDtypeStruct((_B, 16), jnp.float32),
    mesh=plsc.VectorSubcoreMesh(core_axis_name="c", subcore_axis_name="s"),
    compiler_params=pltpu.CompilerParams(needs_layout_passes=False),
    scratch_types=[
        pltpu.VMEM((_P,), jnp.int32),
        pltpu.VMEM((4096,), jnp.int32),
        pltpu.VMEM((256,), jnp.int32),
        pltpu.VMEM((_B,), jnp.int32),
        pltpu.VMEM((16,), jnp.float32),
    ],
)


@jax.jit
def kernel(confidence, predicted_locations, labels, gt_locations):
    B, P, C = confidence.shape
    c0 = confidence[:, :, 0].reshape(B, 1, P)
    c1 = confidence[:, :, 1].reshape(B, 1, P)
    c2 = confidence[:, :, 2].reshape(B, 1, P)
    lab = labels.reshape(B, 1, P)
    lab4 = jnp.repeat(labels, 4, axis=1).reshape(B, 1, 4 * P)
    pd4 = predicted_locations.reshape(B, 1, 4 * P)
    gt4 = gt_locations.reshape(B, 1, 4 * P)

    row_spec = pl.BlockSpec((1, 1, P), lambda i: (i, 0, 0))
    loc_spec = pl.BlockSpec((1, 1, 4 * P), lambda i: (i, 0, 0))
    stat_spec = pl.BlockSpec((1, 1, 128), lambda i: (i, 0, 0))

    keys3, npos3, pnll3, sl13 = pl.pallas_call(
        _tc_row_kernel,
        grid=(B,),
        in_specs=[row_spec, row_spec, row_spec, row_spec, loc_spec,
                  loc_spec, loc_spec],
        out_specs=[row_spec, stat_spec, stat_spec, stat_spec],
        out_shape=[jax.ShapeDtypeStruct((B, 1, P), jnp.int32),
                   jax.ShapeDtypeStruct((B, 1, 128), jnp.int32),
                   jax.ShapeDtypeStruct((B, 1, 128), jnp.float32),
                   jax.ShapeDtypeStruct((B, 1, 128), jnp.float32)],
    )(c0, c1, c2, lab, lab4, pd4, gt4)

    npv = npos3[:, 0, 0]
    negrows = _sc_topk(keys3.reshape(B, P), npv)[:, 0]

    np_tot = jnp.sum(npv).astype(jnp.float32)
    sl1_out = jnp.sum(sl13[:, 0, 0]) / np_tot
    cls_out = (jnp.sum(pnll3[:, 0, 0]) + 0.2 * jnp.sum(negrows)) / np_tot
    return (sl1_out, cls_out)


# speedup vs baseline: 2.0053x; 2.0053x over previous
import functools

import jax
import jax.numpy as jnp
import numpy as np
from jax import lax
from jax.experimental import pallas as pl
from jax.experimental.pallas import tpu as pltpu
from jax.experimental.pallas import tpu_sc as plsc

_B, _P, _C = 32, 20000, 3
_MSB = np.int32(-2147483648)
_NC, _NS, _L = 2, 16, 16
_VECS = _P // _L


def _tc_row_kernel(c0_ref, c1_ref, c2_ref, lab_ref, pd_ref, gt_ref,
                   key_ref, npos_ref, pnll_ref, sl1_ref):
    c0 = c0_ref[0, 0, :]
    c1 = c1_ref[0, 0, :]
    c2 = c2_ref[0, 0, :]
    lab = lab_ref[0, 0, :]

    m = jnp.maximum(c0, jnp.maximum(c1, c2))
    lse = m + jnp.log(jnp.exp(c0 - m) + jnp.exp(c1 - m) + jnp.exp(c2 - m))
    mining = lse - c0
    pos = lab > 0

    bits = lax.bitcast_convert_type(mining, jnp.int32)
    u = jnp.where(bits >= 0, bits | _MSB, ~bits)
    u = jnp.where(pos, np.int32(0), u)
    key_ref[...] = u.reshape(1, 1, _P)

    npos = jnp.sum(pos.astype(jnp.int32))
    sel = jnp.where(lab == 1, c1, c2)
    pnll = jnp.sum(jnp.where(pos, lse - sel, 0.0))

    d = pd_ref[0] - gt_ref[0]
    ad = jnp.abs(d)
    sl1 = jnp.where(ad < 1.0, 0.5 * d * d, ad - 0.5)
    sl1_row = jnp.sum(jnp.where(pos[None, :], sl1, 0.0))

    npos_ref[...] = jnp.full((1, 1, 128), npos, jnp.int32)
    pnll_ref[...] = jnp.full((1, 1, 128), pnll, jnp.float32)
    sl1_ref[...] = jnp.full((1, 1, 128), sl1_row, jnp.float32)


def _sc_topk_body(keys_hbm, npos_hbm, out_hbm,
                  keys_v, hc_v, cc_v, np_v, out_v):
    wid = lax.axis_index("s") * _NC + lax.axis_index("c")
    pltpu.sync_copy(keys_hbm.at[wid], keys_v)
    pltpu.sync_copy(npos_hbm, np_v)

    li = lax.iota(jnp.int32, 16)
    lane = wid % 16
    sel = jnp.where(wid >= 16, np_v[pl.ds(16, 16)], np_v[pl.ds(0, 16)])
    npos = jnp.sum(jnp.where(li == lane, sel, 0))
    k0 = jnp.minimum(3 * npos, _P - npos)

    ones = jnp.ones((16,), jnp.int32)
    zi = jnp.zeros((16,), jnp.int32)
    zf = jnp.zeros((16,), jnp.float32)

    def decode(v):
        return plsc.bitcast(jnp.where(v < 0, v & np.int32(0x7FFFFFFF), ~v),
                            jnp.float32)

    r = k0
    pfx = jnp.int32(0)

    for lvl in range(4):
        shift = 24 - 8 * lvl
        mhi = np.int32(0) if lvl == 0 else np.int32(
            np.uint32(0xFFFFFFFF) << np.uint32(shift + 8))

        def zero4(j, c):
            for q in range(4):
                jj = j * 4 + q
                hc_v[pl.ds(jj * 16, 16)] = zi
            return c
        lax.fori_loop(0, 64, zero4, 0)
        for g in range(16):
            cc_v[pl.ds(g * 16, 16)] = zi

        def dbody5(j, c, pfx=pfx, mhi=mhi, shift=shift, lvl=lvl):
            for q in range(5):
                v = keys_v[pl.ds((j * 5 + q) * 16, 16)]
                digit = lax.shift_right_logical(v, shift) & np.int32(255)
                idx = li * 256 + digit
                if lvl == 0:
                    plsc.addupdate_scatter(hc_v, [idx], ones)
                else:
                    msk = (v & mhi) == pfx
                    plsc.addupdate_scatter(hc_v, [idx], ones, mask=msk)
            return c
        lax.fori_loop(0, _VECS // 5, dbody5, 0)

        def fold2(j, c):
            for q in range(2):
                jj = j * 2 + q
                bidx = (jj & 15) * 16 + li
                plsc.addupdate_scatter(cc_v, [bidx], hc_v[pl.ds(jj * 16, 16)])
            return c
        lax.fori_loop(0, 128, fold2, 0)

        cgs = [cc_v[pl.ds(g * 16, 16)] for g in range(16)]
        tots = [jnp.sum(cg) for cg in cgs]
        hsum = jnp.int32(0)
        highs = [None] * 16
        for g in range(15, -1, -1):
            highs[g] = hsum
            hsum = hsum + tots[g]
        ntrue = jnp.int32(0)
        for g in range(16):
            a_incl = jnp.flip(jnp.cumsum(jnp.flip(cgs[g], 0)), 0)
            a = a_incl - cgs[g] + highs[g]
            ntrue = ntrue + jnp.sum(jnp.where(a < r, 1, 0))
        beta = np.int32(256) - ntrue
        cnt_above = jnp.int32(0)
        for g in range(16):
            mgt = (g * 16 + li) > beta
            cnt_above = cnt_above + jnp.sum(jnp.where(mgt, cgs[g], 0))
        r = r - cnt_above
        pfx = pfx | lax.shift_left(beta, np.int32(shift))

    tx = pfx ^ _MSB
    def sbody5(j, acc):
        sa = acc
        for q in range(5):
            v = keys_v[pl.ds((j * 5 + q) * 16, 16)]
            gt_m = (v ^ _MSB) > tx
            sa = sa + jnp.where(gt_m, decode(v), 0.0)
        return sa
    svec = lax.fori_loop(0, _VECS // 5, sbody5, zf)
    sum_gt = jnp.sum(svec)

    tv = jnp.full((16,), pfx, jnp.int32)
    ans = sum_gt + r.astype(jnp.float32) * decode(tv)
    ans = jnp.where(k0 > 0, ans, 0.0)
    out_v[...] = ans
    pltpu.sync_copy(out_v, out_hbm.at[wid])


_sc_topk = pl.kernel(
    _sc_topk_body,
    out_type=jax.ShapeDtypeStruct((_B, 16), jnp.float32),
    mesh=plsc.VectorSubcoreMesh(core_axis_name="c", subcore_axis_name="s"),
    compiler_params=pltpu.CompilerParams(needs_layout_passes=False),
    scratch_types=[
        pltpu.VMEM((_P,), jnp.int32),
        pltpu.VMEM((4096,), jnp.int32),
        pltpu.VMEM((256,), jnp.int32),
        pltpu.VMEM((_B,), jnp.int32),
        pltpu.VMEM((16,), jnp.float32),
    ],
)


@jax.jit
def kernel(confidence, predicted_locations, labels, gt_locations):
    B, P, C = confidence.shape
    c0 = confidence[:, :, 0].reshape(B, 1, P)
    c1 = confidence[:, :, 1].reshape(B, 1, P)
    c2 = confidence[:, :, 2].reshape(B, 1, P)
    lab = labels.reshape(B, 1, P)
    pdT = predicted_locations.transpose(0, 2, 1)
    gtT = gt_locations.transpose(0, 2, 1)

    row_spec = pl.BlockSpec((1, 1, P), lambda i: (i, 0, 0))
    loc_spec = pl.BlockSpec((1, 4, P), lambda i: (i, 0, 0))
    stat_spec = pl.BlockSpec((1, 1, 128), lambda i: (i, 0, 0))

    keys3, npos3, pnll3, sl13 = pl.pallas_call(
        _tc_row_kernel,
        grid=(B,),
        in_specs=[row_spec, row_spec, row_spec, row_spec, loc_spec, loc_spec],
        out_specs=[row_spec, stat_spec, stat_spec, stat_spec],
        out_shape=[jax.ShapeDtypeStruct((B, 1, P), jnp.int32),
                   jax.ShapeDtypeStruct((B, 1, 128), jnp.int32),
                   jax.ShapeDtypeStruct((B, 1, 128), jnp.float32),
                   jax.ShapeDtypeStruct((B, 1, 128), jnp.float32)],
    )(c0, c1, c2, lab, pdT, gtT)

    npv = npos3[:, 0, 0]
    negrows = _sc_topk(keys3.reshape(B, P), npv)[:, 0]

    np_tot = jnp.sum(npv).astype(jnp.float32)
    sl1_out = jnp.sum(sl13[:, 0, 0]) / np_tot
    cls_out = (jnp.sum(pnll3[:, 0, 0]) + 0.2 * jnp.sum(negrows)) / np_tot
    return (sl1_out, cls_out)
